# baseline (speedup 1.0000x reference)
import functools

import jax
import jax.numpy as jnp
from jax import lax
from jax.experimental import pallas as pl
from jax.experimental.pallas import tpu as pltpu
from jax.experimental.pallas import tpu_sc as plsc

_LANES = 16
_NUM_WORKERS = 32
_NUM_CORES = 2
_NBUF = 4
_CHUNK_S = 4


def _emb_body(batch, s_per_w, embed, seq_len,
              ids_hbm, pos_hbm, table_hbm, out_hbm,
              idx_v, pos_v, *rest):
    bufs = list(rest[:_NBUF])
    gsems = list(rest[_NBUF:2 * _NBUF])
    ssems = list(rest[2 * _NBUF:3 * _NBUF])
    isem, psem = rest[3 * _NBUF], rest[3 * _NBUF + 1]

    wid = lax.axis_index("s") * _NUM_CORES + lax.axis_index("c")
    base_s = wid * s_per_w
    nchunks = s_per_w // _CHUNK_S
    rounds = nchunks // _NBUF
    crows = batch * _CHUNK_S

    id_copy = pltpu.async_copy(ids_hbm.at[wid], idx_v, isem)
    pos_copy = pltpu.async_copy(pos_hbm.at[pl.ds(base_s, s_per_w)], pos_v, psem)
    id_copy.wait()

    def gather_desc(r, j):
        return pltpu.make_async_copy(
            table_hbm.at[idx_v.at[r * _NBUF + j]], bufs[j], gsems[j])

    def store_descs(r, j):
        k = r * _NBUF + j
        return [
            pltpu.make_async_copy(
                bufs[j].at[pl.ds(b * _CHUNK_S, _CHUNK_S)],
                out_hbm.at[pl.ds(b * seq_len + base_s + k * _CHUNK_S, _CHUNK_S)],
                ssems[j])
            for b in range(batch)
        ]

    gather_desc(0, 0).start()
    gather_desc(0, 1).start()
    pos_copy.wait()

    def round_body(r, carry):
        for j in range(_NBUF):
            k = r * _NBUF + j
            gather_desc(r, j).wait()

            @plsc.parallel_loop(0, _CHUNK_S, 1)
            def _add_row(i):
                @plsc.parallel_loop(0, embed, _LANES, unroll=4)
                def _add_col(c):
                    sl = pl.ds(c, _LANES)
                    p = pos_v[k * _CHUNK_S + i, sl]
                    for b in range(batch):
                        plsc.addupdate(bufs[j].at[b * _CHUNK_S + i, sl], p)

            for d in store_descs(r, j):
                d.start()

            j2 = (j + 2) % _NBUF
            if j < _NBUF - 2:
                @pl.when(r > 0)
                def _drain():
                    for d in store_descs(r - 1, j2):
                        d.wait()
                gather_desc(r, j2).start()
            else:
                @pl.when(r < rounds - 1)
                def _drain_and_issue():
                    for d in store_descs(r, j2):
                        d.wait()
                    gather_desc(r + 1, j2).start()
        return carry

    lax.fori_loop(0, rounds, round_body, 0)

    for j in range(_NBUF):
        for d in store_descs(rounds - 1, j):
            d.wait()


def kernel(token_ids, token_table, pos_table):
    batch, seq_len = token_ids.shape
    vocab, embed = token_table.shape
    s_per_w = seq_len // _NUM_WORKERS
    nchunks = s_per_w // _CHUNK_S

    ids = (token_ids.astype(jnp.int32)
           .reshape(batch, _NUM_WORKERS, nchunks, _CHUNK_S)
           .transpose(1, 2, 0, 3)
           .reshape(_NUM_WORKERS, nchunks, batch * _CHUNK_S))

    grid_kernel = functools.partial(
        pl.kernel,
        mesh=plsc.VectorSubcoreMesh(core_axis_name="c", subcore_axis_name="s"),
        out_type=jax.ShapeDtypeStruct((batch * seq_len, embed), jnp.float32),
        scratch_types=(
            [pltpu.VMEM((nchunks, batch * _CHUNK_S), jnp.int32),
             pltpu.VMEM((s_per_w, embed), jnp.float32)]
            + [pltpu.VMEM((batch * _CHUNK_S, embed), jnp.float32)
               for _ in range(_NBUF)]
            + [pltpu.SemaphoreType.DMA for _ in range(2 * _NBUF + 2)]
        ),
    )
    body = grid_kernel(functools.partial(_emb_body, batch, s_per_w, embed, seq_len))
    out = body(ids, pos_table, token_table)
    return out.reshape(batch, seq_len, embed)


# speedup vs baseline: 1.0037x; 1.0037x over previous
import functools

import jax
import jax.numpy as jnp
from jax import lax
from jax.experimental import pallas as pl
from jax.experimental.pallas import tpu as pltpu
from jax.experimental.pallas import tpu_sc as plsc

_LANES = 16
_NUM_WORKERS = 32
_NUM_CORES = 2
_NBUF = 4
_CHUNK_S = 4


def _emb_body(batch, s_per_w, embed, seq_len,
              ids_hbm, pos_hbm, table_hbm, out_hbm,
              idx_v, pos_v, *rest):
    bufs = list(rest[:_NBUF])
    gsems = list(rest[_NBUF:2 * _NBUF])
    ssems = list(rest[2 * _NBUF:3 * _NBUF])
    isem, psem = rest[3 * _NBUF], rest[3 * _NBUF + 1]

    wid = lax.axis_index("s") * _NUM_CORES + lax.axis_index("c")
    base_s = wid * s_per_w
    nchunks = s_per_w // _CHUNK_S
    rounds = nchunks // _NBUF

    id_copy = pltpu.async_copy(ids_hbm.at[wid], idx_v, isem)
    pos_copy = pltpu.async_copy(pos_hbm.at[pl.ds(base_s, s_per_w)], pos_v, psem)
    id_copy.wait()

    def gather_desc(r, j):
        return pltpu.make_async_copy(
            table_hbm.at[idx_v.at[r * _NBUF + j]], bufs[j], gsems[j])

    def store_descs(r, j):
        k = r * _NBUF + j
        return [
            pltpu.make_async_copy(
                bufs[j].at[pl.ds(b * _CHUNK_S, _CHUNK_S)],
                out_hbm.at[pl.ds(b * seq_len + base_s + k * _CHUNK_S, _CHUNK_S)],
                ssems[j])
            for b in range(batch)
        ]

    gather_desc(0, 0).start()
    gather_desc(0, 1).start()
    pos_copy.wait()

    def round_body(r, carry):
        for j in range(_NBUF):
            k = r * _NBUF + j
            gather_desc(r, j).wait()

            @plsc.parallel_loop(0, _CHUNK_S, 1)
            def _add_row(i):
                @plsc.parallel_loop(0, embed, _LANES, unroll=4)
                def _add_col(c):
                    sl = pl.ds(c, _LANES)
                    p = pos_v[k * _CHUNK_S + i, sl]
                    for b in range(batch):
                        plsc.addupdate(bufs[j].at[b * _CHUNK_S + i, sl], p)

            for d in store_descs(r, j):
                d.start()

            j2 = (j + 2) % _NBUF
            if j < _NBUF - 2:
                @pl.when(r > 0)
                def _drain():
                    for d in store_descs(r - 1, j2):
                        d.wait()
                gather_desc(r, j2).start()
            else:
                @pl.when(r < rounds - 1)
                def _drain_and_issue():
                    for d in store_descs(r, j2):
                        d.wait()
                    gather_desc(r + 1, j2).start()
        return carry

    lax.fori_loop(0, rounds, round_body, 0)

    for j in range(_NBUF):
        for d in store_descs(rounds - 1, j):
            d.wait()


def kernel(token_ids, token_table, pos_table):
    batch, seq_len = token_ids.shape
    vocab, embed = token_table.shape
    s_per_w = seq_len // _NUM_WORKERS
    nchunks = s_per_w // _CHUNK_S

    ids = (token_ids.astype(jnp.int32)
           .reshape(batch, _NUM_WORKERS, nchunks, _CHUNK_S)
           .transpose(1, 2, 0, 3)
           .reshape(_NUM_WORKERS, nchunks, batch * _CHUNK_S))

    grid_kernel = functools.partial(
        pl.kernel,
        mesh=plsc.VectorSubcoreMesh(core_axis_name="c", subcore_axis_name="s"),
        out_type=jax.ShapeDtypeStruct((batch * seq_len, embed), jnp.float32),
        scratch_types=(
            [pltpu.VMEM((nchunks, batch * _CHUNK_S), jnp.int32),
             pltpu.VMEM((s_per_w, embed), jnp.float32)]
            + [pltpu.VMEM((batch * _CHUNK_S, embed), jnp.float32)
               for _ in range(_NBUF)]
            + [pltpu.SemaphoreType.DMA for _ in range(2 * _NBUF + 2)]
        ),
    )
    body = grid_kernel(functools.partial(_emb_body, batch, s_per_w, embed, seq_len))
    out = body(ids, pos_table, token_table)
    return out.reshape(batch, seq_len, embed)
